# Initial kernel scaffold; baseline (speedup 1.0000x reference)
import jax
import jax.numpy as jnp
from jax.experimental import pallas as pl


def kernel(x_breaker, x_bus, params, edge_index_bus_breaker, edge_index_breaker_bus):
    raise NotImplementedError("write your pallas kernel here")


# speedup vs baseline: 1.0190x; 1.0190x over previous
import functools

import jax
import jax.numpy as jnp
import numpy as np
from jax import lax
from jax.experimental import pallas as pl
from jax.experimental.pallas import tpu as pltpu
from jax.experimental.pallas import tpu_sc as plsc

_TYPES = ('breaker', 'bus')
_H = 128
_HEADS = 8
_HD = 16
_N = 25000
_E = 300000
_ROWS = 1000
_GRID = _N // _ROWS
_INV_SQRT_HD = 1.0 / np.sqrt(_HD)


def _proj_body(x_ref, w_ref, b_ref, s_ref, t_ref, y_ref, x_out_ref,
               *, out_relu, prologue, emit_x):
    x = x_ref[...]
    if prologue:
        x = jnp.maximum(x * s_ref[...] + t_ref[...], 0.0)
    if emit_x:
        x_out_ref[...] = x
    y = jnp.dot(x, w_ref[...], preferred_element_type=jnp.float32) + b_ref[...]
    if out_relu:
        y = jnp.maximum(y, 0.0)
    y_ref[...] = y


def _proj(x, w, b, scale=None, shift=None, out_relu=False, emit_x=False):
    m = w.shape[1]
    prologue = scale is not None
    if not prologue:
        scale = jnp.zeros((1, _H), jnp.float32)
        shift = jnp.zeros((1, _H), jnp.float32)
    out_shape = [jax.ShapeDtypeStruct((_N, m), jnp.float32)]
    out_specs = [pl.BlockSpec((_ROWS, m), lambda i: (i, 0))]
    if emit_x:
        out_shape.append(jax.ShapeDtypeStruct((_N, _H), jnp.float32))
        out_specs.append(pl.BlockSpec((_ROWS, _H), lambda i: (i, 0)))
    body = functools.partial(_proj_body, out_relu=out_relu,
                             prologue=prologue, emit_x=emit_x)

    def wrapped(x_ref, w_ref, b_ref, s_ref, t_ref, *outs):
        y_ref = outs[0]
        x_out_ref = outs[1] if emit_x else None
        body(x_ref, w_ref, b_ref, s_ref, t_ref, y_ref, x_out_ref)

    res = pl.pallas_call(
        wrapped,
        grid=(_GRID,),
        in_specs=[
            pl.BlockSpec((_ROWS, _H), lambda i: (i, 0)),
            pl.BlockSpec((_H, m), lambda i: (0, 0)),
            pl.BlockSpec((1, m), lambda i: (0, 0)),
            pl.BlockSpec((1, _H), lambda i: (0, 0)),
            pl.BlockSpec((1, _H), lambda i: (0, 0)),
        ],
        out_specs=out_specs,
        out_shape=out_shape,
    )(x, w, b.reshape(1, m), scale, shift)
    if emit_x:
        return res[0], res[1]
    return res[0]


def _post_body(num_ref, den_ref, e8_ref, aw_ref, ab_ref, gam_ref, x_ref,
               t_ref, s_ref, ss_ref):
    den = jnp.dot(den_ref[...], e8_ref[...], preferred_element_type=jnp.float32)
    o = num_ref[...] / (den + 1e-16)
    o = jax.nn.gelu(o)
    t = (jnp.dot(o, aw_ref[...], preferred_element_type=jnp.float32)
         + ab_ref[...] + x_ref[...] * gam_ref[...])
    t_ref[...] = t

    @pl.when(pl.program_id(0) == 0)
    def _():
        s_ref[...] = jnp.zeros_like(s_ref)
        ss_ref[...] = jnp.zeros_like(ss_ref)

    s_ref[...] += jnp.sum(t, axis=0, keepdims=True)
    ss_ref[...] += jnp.sum(t * t, axis=0, keepdims=True)


def _post(num, den, x_prev, a_w, a_b, beta):
    e8 = jnp.repeat(jnp.eye(_HEADS, dtype=jnp.float32), _HD, axis=1)
    aw_eff = a_w * beta
    ab_eff = (a_b * beta).reshape(1, _H)
    gam = jnp.broadcast_to((1.0 - beta).reshape(1, 1), (1, _H))
    t, s, ss = pl.pallas_call(
        _post_body,
        grid=(_GRID,),
        in_specs=[
            pl.BlockSpec((_ROWS, _H), lambda i: (i, 0)),
            pl.BlockSpec((_ROWS, _HEADS), lambda i: (i, 0)),
            pl.BlockSpec((_HEADS, _H), lambda i: (0, 0)),
            pl.BlockSpec((_H, _H), lambda i: (0, 0)),
            pl.BlockSpec((1, _H), lambda i: (0, 0)),
            pl.BlockSpec((1, _H), lambda i: (0, 0)),
            pl.BlockSpec((_ROWS, _H), lambda i: (i, 0)),
        ],
        out_specs=[
            pl.BlockSpec((_ROWS, _H), lambda i: (i, 0)),
            pl.BlockSpec((1, _H), lambda i: (0, 0)),
            pl.BlockSpec((1, _H), lambda i: (0, 0)),
        ],
        out_shape=[
            jax.ShapeDtypeStruct((_N, _H), jnp.float32),
            jax.ShapeDtypeStruct((1, _H), jnp.float32),
            jax.ShapeDtypeStruct((1, _H), jnp.float32),
        ],
    )(num, den, e8, aw_eff, ab_eff, gam, x_prev)
    return t, s, ss


def _bn_affine(s, ss, g, b):
    mean = s / _N
    var = ss / _N - mean * mean
    scale = (g.reshape(1, _H)) * lax.rsqrt(var + 1e-5)
    shift = b.reshape(1, _H) - mean * scale
    return scale, shift


def _edge_op(q_d, k_rel, v_rel, src, dst, p_rel):
    qh = q_d.reshape(_N, _HEADS, _HD)
    kh = k_rel.reshape(_N, _HEADS, _HD)
    vh = v_rel.reshape(_N, _HEADS, _HD)
    s = (qh[dst] * kh[src]).sum(-1) * (p_rel * _INV_SQRT_HD)
    m = jax.ops.segment_max(s, dst, num_segments=_N)
    m = jnp.where(jnp.isfinite(m), m, 0.0)
    e = jnp.exp(s - m[dst])
    den = jax.ops.segment_sum(e, dst, num_segments=_N)
    num = jax.ops.segment_sum(vh[src] * e[:, :, None], dst, num_segments=_N)
    return num.reshape(_N, _H), den


def _bd(w):
    out = jnp.zeros((_H, _H), w.dtype)
    for h in range(_HEADS):
        out = out.at[h * _HD:(h + 1) * _HD, h * _HD:(h + 1) * _HD].set(w[h])
    return out


def kernel(x_breaker, x_bus, params, edge_index_bus_breaker,
           edge_index_breaker_bus):
    p = params
    src_bb = edge_index_bus_breaker[0].astype(jnp.int32)
    dst_bb = edge_index_bus_breaker[1].astype(jnp.int32)
    src_brb = edge_index_breaker_bus[0].astype(jnp.int32)
    dst_brb = edge_index_breaker_bus[1].astype(jnp.int32)

    ek = {'bus': 'bus__connects__breaker', 'breaker': 'breaker__connects__bus'}

    def fused_kv(l, t):
        key = f'{l}_{ek[t]}'
        bd_att = _bd(p['W_att_' + key])
        bd_msg = _bd(p['W_msg_' + key])
        kw = p[f'K_w_{l}_{t}'] @ bd_att
        kb = p[f'K_b_{l}_{t}'] @ bd_att
        vw = p[f'V_w_{l}_{t}'] @ bd_msg
        vb = p[f'V_b_{l}_{t}'] @ bd_msg
        return kw, kb, vw, vb

    h = {t: _proj(x_breaker if t == 'breaker' else x_bus,
                  p['lin_w_' + t], p['lin_b_' + t], out_relu=True)
         for t in _TYPES}

    qkv = {}
    for t in _TYPES:
        kw, kb, vw, vb = fused_kv(0, t)
        wcat = jnp.concatenate([p[f'Q_w_0_{t}'], kw, vw], axis=1)
        bcat = jnp.concatenate([p[f'Q_b_0_{t}'], kb, vb], axis=0)
        qkv[t] = _proj(h[t], wcat, bcat)

    num_br, den_br = _edge_op(
        qkv['breaker'][:, :_H], qkv['bus'][:, _H:2 * _H],
        qkv['bus'][:, 2 * _H:], src_bb, dst_bb,
        p['p_rel_0_bus__connects__breaker'])
    num_bus, den_bus = _edge_op(
        qkv['bus'][:, :_H], qkv['breaker'][:, _H:2 * _H],
        qkv['breaker'][:, 2 * _H:], src_brb, dst_brb,
        p['p_rel_0_breaker__connects__bus'])

    beta = {t: jax.nn.sigmoid(p[f'skip_0_{t}']) for t in _TYPES}
    t_br, s_br, ss_br = _post(num_br, den_br, h['breaker'],
                              p['A_w_0_breaker'], p['A_b_0_breaker'],
                              beta['breaker'])
    t_bus, s_bus, ss_bus = _post(num_bus, den_bus, h['bus'],
                                 p['A_w_0_bus'], p['A_b_0_bus'], beta['bus'])
    sc_br, sh_br = _bn_affine(s_br, ss_br, p['bn_g_0_breaker'], p['bn_b_0_breaker'])
    sc_bus, sh_bus = _bn_affine(s_bus, ss_bus, p['bn_g_0_bus'], p['bn_b_0_bus'])

    q1_br, x1_br = _proj(t_br, p['Q_w_1_breaker'], p['Q_b_1_breaker'],
                         scale=sc_br, shift=sh_br, emit_x=True)
    kw, kb, vw, vb = fused_kv(1, 'bus')
    kv1_bus = _proj(t_bus, jnp.concatenate([kw, vw], axis=1),
                    jnp.concatenate([kb, vb], axis=0),
                    scale=sc_bus, shift=sh_bus)

    num1, den1 = _edge_op(q1_br, kv1_bus[:, :_H], kv1_bus[:, _H:],
                          src_bb, dst_bb, p['p_rel_1_bus__connects__breaker'])

    beta1 = jax.nn.sigmoid(p['skip_1_breaker'])
    t1_br, s1, ss1 = _post(num1, den1, x1_br, p['A_w_1_breaker'],
                           p['A_b_1_breaker'], beta1)
    sc1, sh1 = _bn_affine(s1, ss1, p['bn_g_1_breaker'], p['bn_b_1_breaker'])

    out = _proj(t1_br, p['final_w'], p['final_b'], scale=sc1, shift=sh1)
    return out


# speedup vs baseline: 12.1686x; 11.9420x over previous
import functools

import jax
import jax.numpy as jnp
import numpy as np
from jax import lax
from jax.experimental import pallas as pl
from jax.experimental.pallas import tpu as pltpu
from jax.experimental.pallas import tpu_sc as plsc

_TYPES = ('breaker', 'bus')
_H = 128
_HEADS = 8
_HD = 16
_N = 25000
_E = 300000
_ROWS = 1000
_GRID = _N // _ROWS
_INV_SQRT_HD = 1.0 / np.sqrt(_HD)


def _proj(x, w, b, scale=None, shift=None, out_relu=False, emit_x=False,
          want_src=False, want_q=False):
    m = w.shape[1]
    prologue = scale is not None
    if not prologue:
        scale = jnp.zeros((1, _H), jnp.float32)
        shift = jnp.zeros((1, _H), jnp.float32)
    out_shape, out_specs = [], []
    if want_src:
        out_shape.append(jax.ShapeDtypeStruct((2, _N, 128), jnp.float32))
        out_specs.append(pl.BlockSpec((2, _ROWS, 128), lambda i: (0, i, 0)))
    if want_q:
        out_shape.append(jax.ShapeDtypeStruct((_N, 128), jnp.float32))
        out_specs.append(pl.BlockSpec((_ROWS, 128), lambda i: (i, 0)))
    if not (want_src or want_q):
        out_shape.append(jax.ShapeDtypeStruct((_N, m), jnp.float32))
        out_specs.append(pl.BlockSpec((_ROWS, m), lambda i: (i, 0)))
    if emit_x:
        out_shape.append(jax.ShapeDtypeStruct((_N, _H), jnp.float32))
        out_specs.append(pl.BlockSpec((_ROWS, _H), lambda i: (i, 0)))

    def body(x_ref, w_ref, b_ref, s_ref, t_ref, *outs):
        x_blk = x_ref[...]
        if prologue:
            x_blk = jnp.maximum(x_blk * s_ref[...] + t_ref[...], 0.0)
        if emit_x:
            outs[-1][...] = x_blk
        y = jnp.dot(x_blk, w_ref[...],
                    preferred_element_type=jnp.float32) + b_ref[...]
        if out_relu:
            y = jnp.maximum(y, 0.0)
        o = 0
        col = 0
        if want_src:
            outs[o][0] = y[:, :128]
            outs[o][1] = y[:, 128:256]
            o += 1
            col = 256
        if want_q:
            outs[o][...] = y[:, col:col + 128]
            o += 1
        if not (want_src or want_q):
            outs[0][...] = y

    res = pl.pallas_call(
        body,
        grid=(_GRID,),
        in_specs=[
            pl.BlockSpec((_ROWS, _H), lambda i: (i, 0)),
            pl.BlockSpec((_H, m), lambda i: (0, 0)),
            pl.BlockSpec((1, m), lambda i: (0, 0)),
            pl.BlockSpec((1, _H), lambda i: (0, 0)),
            pl.BlockSpec((1, _H), lambda i: (0, 0)),
        ],
        out_specs=out_specs,
        out_shape=out_shape,
    )(x, w, b.reshape(1, m), scale, shift)
    return res if len(res) > 1 else res[0]


def _post_body(num_ref, den_ref, e8_ref, aw_ref, ab_ref, gam_ref, x_ref,
               t_ref, s_ref, ss_ref):
    den = jnp.dot(den_ref[...], e8_ref[...], preferred_element_type=jnp.float32)
    o = num_ref[...] / (den + 1e-16)
    o = jax.nn.gelu(o)
    t = (jnp.dot(o, aw_ref[...], preferred_element_type=jnp.float32)
         + ab_ref[...] + x_ref[...] * gam_ref[...])
    t_ref[...] = t

    @pl.when(pl.program_id(0) == 0)
    def _():
        s_ref[...] = jnp.zeros_like(s_ref)
        ss_ref[...] = jnp.zeros_like(ss_ref)

    s_ref[...] += jnp.sum(t, axis=0, keepdims=True)
    ss_ref[...] += jnp.sum(t * t, axis=0, keepdims=True)


def _post(num, den, x_prev, a_w, a_b, beta):
    e8 = jnp.repeat(jnp.eye(_HEADS, dtype=jnp.float32), _HD, axis=1)
    aw_eff = a_w * beta
    ab_eff = (a_b * beta).reshape(1, _H)
    gam = jnp.broadcast_to((1.0 - beta).reshape(1, 1), (1, _H))
    t, s, ss = pl.pallas_call(
        _post_body,
        grid=(_GRID,),
        in_specs=[
            pl.BlockSpec((_ROWS, _H), lambda i: (i, 0)),
            pl.BlockSpec((_ROWS, _HEADS), lambda i: (i, 0)),
            pl.BlockSpec((_HEADS, _H), lambda i: (0, 0)),
            pl.BlockSpec((_H, _H), lambda i: (0, 0)),
            pl.BlockSpec((1, _H), lambda i: (0, 0)),
            pl.BlockSpec((1, _H), lambda i: (0, 0)),
            pl.BlockSpec((_ROWS, _H), lambda i: (i, 0)),
        ],
        out_specs=[
            pl.BlockSpec((_ROWS, _H), lambda i: (i, 0)),
            pl.BlockSpec((1, _H), lambda i: (0, 0)),
            pl.BlockSpec((1, _H), lambda i: (0, 0)),
        ],
        out_shape=[
            jax.ShapeDtypeStruct((_N, _H), jnp.float32),
            jax.ShapeDtypeStruct((1, _H), jnp.float32),
            jax.ShapeDtypeStruct((1, _H), jnp.float32),
        ],
    )(num, den, e8, aw_eff, ab_eff, gam, x_prev)
    return t, s, ss


def _bn_affine(s, ss, g, b):
    mean = s / _N
    var = ss / _N - mean * mean
    scale = (g.reshape(1, _H)) * lax.rsqrt(var + 1e-5)
    shift = b.reshape(1, _H) - mean * scale
    return scale, shift


_B = 48
_NCHUNK = _E // _B
_NUMROWS = 12504
_DENROWS = 784
_RPN = 776
_RPD = 48


def _lanesum(v):
    dnums = lax.GatherDimensionNumbers(offset_dims=(), collapsed_slice_dims=(0,),
                                       start_index_map=(0,))
    for k in (8, 4, 2, 1):
        idx = jnp.bitwise_xor(lax.iota(jnp.int32, 16), k)
        v = v + lax.gather(v, idx[:, None], dnums, slice_sizes=(1,),
                           mode=lax.GatherScatterMode.PROMISE_IN_BOUNDS)
    return v


def _edge_sc_kernel(src_tab, q_tab, src_idx, dst_idx, zrows):
    mesh = plsc.VectorSubcoreMesh(core_axis_name='c', subcore_axis_name='s')

    @functools.partial(
        pl.kernel,
        out_type=[jax.ShapeDtypeStruct((2, _NUMROWS, 128), jnp.float32),
                  jax.ShapeDtypeStruct((2, _DENROWS, 128), jnp.float32)],
        mesh=mesh,
        scratch_types=[
            pltpu.VMEM_SHARED((_NUMROWS, 128), jnp.float32),
            pltpu.VMEM_SHARED((_DENROWS, 128), jnp.float32),
            pltpu.VMEM((_B,), jnp.int32),
            pltpu.VMEM((_B,), jnp.int32),
            pltpu.VMEM((_B,), jnp.int32),
            pltpu.VMEM((_B,), jnp.int32),
            pltpu.VMEM((_B, 128), jnp.float32),
            pltpu.VMEM((_B, 128), jnp.float32),
            pltpu.VMEM((_B, 128), jnp.float32),
        ],
    )
    def k(src_tab_hbm, q_tab_hbm, sidx_hbm, didx_hbm, z_hbm,
          num_out, den_out, num_acc, den_acc,
          sidx_v, didx_v, didx_h, didx_d, srcrows, qrows, wbuf):
        c = lax.axis_index('c')
        s = lax.axis_index('s')

        rn = s * _RPN
        rd = s * _RPD
        pltpu.sync_copy(z_hbm.at[pl.ds(0, _RPN)], num_acc.at[pl.ds(rn, _RPN)])
        pltpu.sync_copy(z_hbm.at[pl.ds(0, _RPD)], den_acc.at[pl.ds(rd, _RPD)])

        @pl.when(s == 15)
        def _():
            pltpu.sync_copy(z_hbm.at[pl.ds(0, 88)],
                            num_acc.at[pl.ds(16 * _RPN, 88)])
            pltpu.sync_copy(z_hbm.at[pl.ds(0, 16)],
                            den_acc.at[pl.ds(16 * _RPD, 16)])

        plsc.subcore_barrier()

        coff = c * _N
        my_n = (_NCHUNK - s + 15) // 16
        iota = lax.iota(jnp.int32, 16)

        def chunk_body(i, carry):
            base = (s + i * 16) * _B
            pltpu.sync_copy(sidx_hbm.at[pl.ds(base, _B)], sidx_v)
            pltpu.sync_copy(didx_hbm.at[pl.ds(base, _B)], didx_v)
            for g in range(_B // 16):
                sl = pl.ds(g * 16, 16)
                sidx_v[sl] = sidx_v[sl] + coff
                dv = didx_v[sl]
                didx_h[sl] = lax.shift_right_logical(dv, 1)
                didx_d[sl] = lax.shift_right_logical(dv, 5)
            pltpu.sync_copy(src_tab_hbm.at[sidx_v], srcrows)
            pltpu.sync_copy(q_tab_hbm.at[didx_v], qrows)

            def group_body(g, carry2):
                dgrp = didx_v[pl.ds(g * 16, 16)]
                for j in range(16):
                    e = g * 16 + j
                    dn = dgrp[j]
                    evf = lax.convert_element_type(1 - (dn & 1), jnp.float32)
                    evvec = jnp.broadcast_to(evf, (16,))
                    odvec = 1.0 - evvec
                    off = (dn & 3) * 4
                    gsel = lax.shift_right_logical(dn & 31, 2)
                    wrow = jnp.zeros((16,), jnp.float32)
                    for h in range(4):
                        qv = qrows[e, pl.ds(c * 64 + h * _HD, _HD)]
                        kv = srcrows[e, pl.ds(h * _HD, _HD)]
                        wv = jnp.exp(_lanesum(qv * kv))
                        vv = srcrows[e, pl.ds(64 + h * _HD, _HD)]
                        val = vv * wv
                        srcrows[e, pl.ds(h * _HD, _HD)] = val * evvec
                        srcrows[e, pl.ds(64 + h * _HD, _HD)] = val * odvec
                        hm = lax.convert_element_type(
                            1 - jnp.minimum(jnp.abs(iota - (off + h)), 1),
                            jnp.float32)
                        wrow = wrow + wv * hm
                    for jj in range(8):
                        eq = 1 - jnp.minimum(jnp.abs(gsel - jj), 1)
                        eqf = lax.convert_element_type(eq, jnp.float32)
                        wbuf[e, pl.ds(jj * _HD, _HD)] = (
                            wrow * jnp.broadcast_to(eqf, (16,)))
                return carry2

            lax.fori_loop(0, _B // 16, group_body, 0)
            pltpu.sync_copy(srcrows, num_acc.at[didx_h], add=True)
            pltpu.sync_copy(wbuf, den_acc.at[didx_d], add=True)
            return carry

        lax.fori_loop(0, my_n, chunk_body, 0)
        plsc.subcore_barrier()

        pltpu.sync_copy(num_acc.at[pl.ds(rn, _RPN)],
                        num_out.at[c, pl.ds(rn, _RPN), :])
        pltpu.sync_copy(den_acc.at[pl.ds(rd, _RPD)],
                        den_out.at[c, pl.ds(rd, _RPD), :])

        @pl.when(s == 15)
        def _():
            pltpu.sync_copy(num_acc.at[pl.ds(16 * _RPN, 88)],
                            num_out.at[c, pl.ds(16 * _RPN, 88), :])
            pltpu.sync_copy(den_acc.at[pl.ds(16 * _RPD, 16)],
                            den_out.at[c, pl.ds(16 * _RPD, 16), :])

    return k(src_tab, q_tab, src_idx, dst_idx, zrows)


def _edge_op(src_tab, q_tab, src, dst, zrows):
    num_p, den_p = _edge_sc_kernel(src_tab, q_tab, src, dst, zrows)
    num0 = num_p[0][:_N // 2].reshape(_N, 64)
    num1 = num_p[1][:_N // 2].reshape(_N, 64)
    den0 = den_p[0][:782].reshape(782 * 32, 4)[:_N]
    den1 = den_p[1][:782].reshape(782 * 32, 4)[:_N]
    num = jnp.concatenate([num0, num1], axis=1)
    den = jnp.concatenate([den0, den1], axis=1)
    return num, den


def _bd(w):
    out = jnp.zeros((_H, _H), w.dtype)
    for h in range(_HEADS):
        out = out.at[h * _HD:(h + 1) * _HD, h * _HD:(h + 1) * _HD].set(w[h])
    return out


def kernel(x_breaker, x_bus, params, edge_index_bus_breaker,
           edge_index_breaker_bus):
    p = params
    src_bb = edge_index_bus_breaker[0].astype(jnp.int32)
    dst_bb = edge_index_bus_breaker[1].astype(jnp.int32)
    src_brb = edge_index_breaker_bus[0].astype(jnp.int32)
    dst_brb = edge_index_breaker_bus[1].astype(jnp.int32)

    ek = {'bus': 'bus__connects__breaker', 'breaker': 'breaker__connects__bus'}

    def fused_kv(l, t):
        key = f'{l}_{ek[t]}'
        bd_att = _bd(p['W_att_' + key])
        bd_msg = _bd(p['W_msg_' + key])
        pscale = jnp.repeat(p['p_rel_' + key] * _INV_SQRT_HD, _HD)
        kw = (p[f'K_w_{l}_{t}'] @ bd_att) * pscale[None, :]
        kb = (p[f'K_b_{l}_{t}'] @ bd_att) * pscale
        vw = p[f'V_w_{l}_{t}'] @ bd_msg
        vb = p[f'V_b_{l}_{t}'] @ bd_msg
        wcat = jnp.concatenate([kw[:, :64], vw[:, :64], kw[:, 64:], vw[:, 64:]], 1)
        bcat = jnp.concatenate([kb[:64], vb[:64], kb[64:], vb[64:]], 0)
        return wcat, bcat

    zrows = jnp.zeros((_RPN, 128), jnp.float32)

    h = {t: _proj(x_breaker if t == 'breaker' else x_bus,
                  p['lin_w_' + t], p['lin_b_' + t], out_relu=True)
         for t in _TYPES}

    tabs = {}
    for t in _TYPES:
        kvw, kvb = fused_kv(0, t)
        wcat = jnp.concatenate([kvw, p[f'Q_w_0_{t}']], axis=1)
        bcat = jnp.concatenate([kvb, p[f'Q_b_0_{t}']], axis=0)
        src_tab, q_tab = _proj(h[t], wcat, bcat, want_src=True, want_q=True)
        tabs[t] = (src_tab.reshape(2 * _N, 128), q_tab)

    num_br, den_br = _edge_op(tabs['bus'][0], tabs['breaker'][1],
                              src_bb, dst_bb, zrows)
    num_bus, den_bus = _edge_op(tabs['breaker'][0], tabs['bus'][1],
                                src_brb, dst_brb, zrows)

    beta = {t: jax.nn.sigmoid(p[f'skip_0_{t}']) for t in _TYPES}
    t_br, s_br, ss_br = _post(num_br, den_br, h['breaker'],
                              p['A_w_0_breaker'], p['A_b_0_breaker'],
                              beta['breaker'])
    t_bus, s_bus, ss_bus = _post(num_bus, den_bus, h['bus'],
                                 p['A_w_0_bus'], p['A_b_0_bus'], beta['bus'])
    sc_br, sh_br = _bn_affine(s_br, ss_br, p['bn_g_0_breaker'], p['bn_b_0_breaker'])
    sc_bus, sh_bus = _bn_affine(s_bus, ss_bus, p['bn_g_0_bus'], p['bn_b_0_bus'])

    q1_tab, x1_br = _proj(t_br, p['Q_w_1_breaker'], p['Q_b_1_breaker'],
                          scale=sc_br, shift=sh_br, want_q=True, emit_x=True)
    kvw, kvb = fused_kv(1, 'bus')
    src1_tab = _proj(t_bus, kvw, kvb, scale=sc_bus, shift=sh_bus,
                     want_src=True)

    num1, den1 = _edge_op(src1_tab.reshape(2 * _N, 128), q1_tab,
                          src_bb, dst_bb, zrows)

    beta1 = jax.nn.sigmoid(p['skip_1_breaker'])
    t1_br, s1, ss1 = _post(num1, den1, x1_br, p['A_w_1_breaker'],
                           p['A_b_1_breaker'], beta1)
    sc1, sh1 = _bn_affine(s1, ss1, p['bn_g_1_breaker'], p['bn_b_1_breaker'])

    out = _proj(t1_br, p['final_w'], p['final_b'], scale=sc1, shift=sh1)
    return out


# speedup vs baseline: 19.7080x; 1.6196x over previous
import functools

import jax
import jax.numpy as jnp
import numpy as np
from jax import lax
from jax.experimental import pallas as pl
from jax.experimental.pallas import tpu as pltpu
from jax.experimental.pallas import tpu_sc as plsc

_TYPES = ('breaker', 'bus')
_H = 128
_HEADS = 8
_HD = 16
_N = 25000
_E = 300000
_ROWS = 1000
_GRID = _N // _ROWS
_INV_SQRT_HD = 1.0 / np.sqrt(_HD)


def _proj(x, w, b, scale=None, shift=None, out_relu=False, emit_x=False,
          want_src=False, want_q=False):
    m = w.shape[1]
    prologue = scale is not None
    if not prologue:
        scale = jnp.zeros((1, _H), jnp.float32)
        shift = jnp.zeros((1, _H), jnp.float32)
    out_shape, out_specs = [], []
    if want_src:
        out_shape.append(jax.ShapeDtypeStruct((2, _N, 128), jnp.float32))
        out_specs.append(pl.BlockSpec((2, _ROWS, 128), lambda i: (0, i, 0)))
    if want_q:
        out_shape.append(jax.ShapeDtypeStruct((_N, 128), jnp.float32))
        out_specs.append(pl.BlockSpec((_ROWS, 128), lambda i: (i, 0)))
    if not (want_src or want_q):
        out_shape.append(jax.ShapeDtypeStruct((_N, m), jnp.float32))
        out_specs.append(pl.BlockSpec((_ROWS, m), lambda i: (i, 0)))
    if emit_x:
        out_shape.append(jax.ShapeDtypeStruct((_N, _H), jnp.float32))
        out_specs.append(pl.BlockSpec((_ROWS, _H), lambda i: (i, 0)))

    def body(x_ref, w_ref, b_ref, s_ref, t_ref, *outs):
        x_blk = x_ref[...]
        if prologue:
            x_blk = jnp.maximum(x_blk * s_ref[...] + t_ref[...], 0.0)
        if emit_x:
            outs[-1][...] = x_blk
        y = jnp.dot(x_blk, w_ref[...],
                    preferred_element_type=jnp.float32) + b_ref[...]
        if out_relu:
            y = jnp.maximum(y, 0.0)
        o = 0
        col = 0
        if want_src:
            outs[o][0] = y[:, :128]
            outs[o][1] = y[:, 128:256]
            o += 1
            col = 256
        if want_q:
            outs[o][...] = y[:, col:col + 128]
            o += 1
        if not (want_src or want_q):
            outs[0][...] = y

    res = pl.pallas_call(
        body,
        grid=(_GRID,),
        in_specs=[
            pl.BlockSpec((_ROWS, _H), lambda i: (i, 0)),
            pl.BlockSpec((_H, m), lambda i: (0, 0)),
            pl.BlockSpec((1, m), lambda i: (0, 0)),
            pl.BlockSpec((1, _H), lambda i: (0, 0)),
            pl.BlockSpec((1, _H), lambda i: (0, 0)),
        ],
        out_specs=out_specs,
        out_shape=out_shape,
    )(x, w, b.reshape(1, m), scale, shift)
    return res if len(res) > 1 else res[0]


def _post_body(num_ref, den_ref, e8_ref, aw_ref, ab_ref, gam_ref, x_ref,
               t_ref, s_ref, ss_ref):
    den = jnp.dot(den_ref[...], e8_ref[...], preferred_element_type=jnp.float32)
    o = num_ref[...] / (den + 1e-16)
    o = jax.nn.gelu(o)
    t = (jnp.dot(o, aw_ref[...], preferred_element_type=jnp.float32)
         + ab_ref[...] + x_ref[...] * gam_ref[...])
    t_ref[...] = t

    @pl.when(pl.program_id(0) == 0)
    def _():
        s_ref[...] = jnp.zeros_like(s_ref)
        ss_ref[...] = jnp.zeros_like(ss_ref)

    s_ref[...] += jnp.sum(t, axis=0, keepdims=True)
    ss_ref[...] += jnp.sum(t * t, axis=0, keepdims=True)


def _post(num, den, x_prev, a_w, a_b, beta):
    e8 = jnp.repeat(jnp.eye(_HEADS, dtype=jnp.float32), _HD, axis=1)
    aw_eff = a_w * beta
    ab_eff = (a_b * beta).reshape(1, _H)
    gam = jnp.broadcast_to((1.0 - beta).reshape(1, 1), (1, _H))
    t, s, ss = pl.pallas_call(
        _post_body,
        grid=(_GRID,),
        in_specs=[
            pl.BlockSpec((_ROWS, _H), lambda i: (i, 0)),
            pl.BlockSpec((_ROWS, _HEADS), lambda i: (i, 0)),
            pl.BlockSpec((_HEADS, _H), lambda i: (0, 0)),
            pl.BlockSpec((_H, _H), lambda i: (0, 0)),
            pl.BlockSpec((1, _H), lambda i: (0, 0)),
            pl.BlockSpec((1, _H), lambda i: (0, 0)),
            pl.BlockSpec((_ROWS, _H), lambda i: (i, 0)),
        ],
        out_specs=[
            pl.BlockSpec((_ROWS, _H), lambda i: (i, 0)),
            pl.BlockSpec((1, _H), lambda i: (0, 0)),
            pl.BlockSpec((1, _H), lambda i: (0, 0)),
        ],
        out_shape=[
            jax.ShapeDtypeStruct((_N, _H), jnp.float32),
            jax.ShapeDtypeStruct((1, _H), jnp.float32),
            jax.ShapeDtypeStruct((1, _H), jnp.float32),
        ],
    )(num, den, e8, aw_eff, ab_eff, gam, x_prev)
    return t, s, ss


def _bn_affine(s, ss, g, b):
    mean = s / _N
    var = ss / _N - mean * mean
    scale = (g.reshape(1, _H)) * lax.rsqrt(var + 1e-5)
    shift = b.reshape(1, _H) - mean * scale
    return scale, shift


_B = 48
_NCHUNK = _E // _B
_NUMROWS = 12504
_DENROWS = 784
_RPN = 776
_RPD = 48


_DNUMS = lax.GatherDimensionNumbers(offset_dims=(), collapsed_slice_dims=(0,),
                                    start_index_map=(0,))


def _vperm(v, idx):
    return lax.gather(v, idx[:, None], _DNUMS, slice_sizes=(1,),
                      mode=lax.GatherScatterMode.PROMISE_IN_BOUNDS)


def _lanesum(v, perms):
    for idx in perms:
        v = v + _vperm(v, idx)
    return v


def _edge_sc_kernel(src_tab, q_tab, src_idx, dst_idx, zrows):
    mesh = plsc.VectorSubcoreMesh(core_axis_name='c', subcore_axis_name='s')

    @functools.partial(
        pl.kernel,
        out_type=[jax.ShapeDtypeStruct((2, _NUMROWS, 128), jnp.float32),
                  jax.ShapeDtypeStruct((2, _DENROWS, 128), jnp.float32)],
        mesh=mesh,
        scratch_types=[
            pltpu.VMEM_SHARED((_NUMROWS, 128), jnp.float32),
            pltpu.VMEM_SHARED((_DENROWS, 128), jnp.float32),
            pltpu.VMEM((_B,), jnp.int32),
            pltpu.VMEM((_B,), jnp.int32),
            pltpu.VMEM((_B,), jnp.int32),
            pltpu.VMEM((_B,), jnp.int32),
            pltpu.VMEM((_B, 128), jnp.float32),
            pltpu.VMEM((_B, 128), jnp.float32),
            pltpu.VMEM((_B, 128), jnp.float32),
            pltpu.SemaphoreType.DMA,
            pltpu.SemaphoreType.DMA,
        ],
    )
    def k(src_tab_hbm, q_tab_hbm, sidx_hbm, didx_hbm, z_hbm,
          num_out, den_out, num_acc, den_acc,
          sidx_v, didx_v, didx_h, didx_d, srcrows, qrows, wbuf,
          sem_a, sem_b):
        c = lax.axis_index('c')
        s = lax.axis_index('s')

        rn = s * _RPN
        rd = s * _RPD
        pltpu.sync_copy(z_hbm.at[pl.ds(0, _RPN)], num_acc.at[pl.ds(rn, _RPN)])
        pltpu.sync_copy(z_hbm.at[pl.ds(0, _RPD)], den_acc.at[pl.ds(rd, _RPD)])

        @pl.when(s == 15)
        def _():
            pltpu.sync_copy(z_hbm.at[pl.ds(0, 88)],
                            num_acc.at[pl.ds(16 * _RPN, 88)])
            pltpu.sync_copy(z_hbm.at[pl.ds(0, 16)],
                            den_acc.at[pl.ds(16 * _RPD, 16)])

        plsc.subcore_barrier()

        coff = c * _N
        my_n = (_NCHUNK - s + 15) // 16
        iota = lax.iota(jnp.int32, 16)

        perms = [jnp.bitwise_xor(iota, kk) for kk in (8, 4, 2, 1)]
        onehot = [lax.convert_element_type(
            1 - jnp.minimum(jnp.abs(iota - h), 1), jnp.float32)
            for h in range(4)]
        zv = jnp.zeros((16,), jnp.float32)

        def chunk_body(i, carry):
            base = (s + i * 16) * _B
            d1 = pltpu.async_copy(sidx_hbm.at[pl.ds(base, _B)], sidx_v, sem_a)
            d2 = pltpu.async_copy(didx_hbm.at[pl.ds(base, _B)], didx_v, sem_b)
            d1.wait()
            d2.wait()
            for g in range(_B // 16):
                sl = pl.ds(g * 16, 16)
                sidx_v[sl] = sidx_v[sl] + coff
                dv = didx_v[sl]
                didx_h[sl] = lax.shift_right_logical(dv, 1)
                didx_d[sl] = lax.shift_right_logical(dv, 5)
            d1 = pltpu.async_copy(src_tab_hbm.at[sidx_v], srcrows, sem_a)
            d2 = pltpu.async_copy(q_tab_hbm.at[didx_v], qrows, sem_b)
            d1.wait()
            d2.wait()

            def group_body(g, carry2):
                dgrp = didx_v[pl.ds(g * 16, 16)]
                for j in range(16):
                    e = g * 16 + j
                    dn = dgrp[j]
                    off64 = (dn & 1) * 64
                    offo = jnp.bitwise_xor(off64, 64)
                    grp16 = (dn & 28) * 4
                    offrot = (dn & 3) * 4
                    wrow = zv
                    for h in range(4):
                        qv = qrows[e, pl.ds(c * 64 + h * _HD, _HD)]
                        kv = srcrows[e, pl.ds(h * _HD, _HD)]
                        wv = jnp.exp(_lanesum(qv * kv, perms))
                        vv = srcrows[e, pl.ds(64 + h * _HD, _HD)]
                        val = vv * wv
                        srcrows[e, pl.ds(off64 + h * _HD, _HD)] = val
                        srcrows[e, pl.ds(offo + h * _HD, _HD)] = zv
                        wrow = wrow + wv * onehot[h]
                    for jj in range(8):
                        wbuf[e, pl.ds(jj * _HD, _HD)] = zv
                    wrot = _vperm(wrow, (iota - offrot) & 15)
                    wbuf[e, pl.ds(grp16, _HD)] = wrot
                return carry2

            lax.fori_loop(0, _B // 16, group_body, 0)
            d1 = pltpu.async_copy(srcrows, num_acc.at[didx_h], sem_a, add=True)
            d2 = pltpu.async_copy(wbuf, den_acc.at[didx_d], sem_b, add=True)
            d1.wait()
            d2.wait()
            return carry

        lax.fori_loop(0, my_n, chunk_body, 0)
        plsc.subcore_barrier()

        pltpu.sync_copy(num_acc.at[pl.ds(rn, _RPN)],
                        num_out.at[c, pl.ds(rn, _RPN), :])
        pltpu.sync_copy(den_acc.at[pl.ds(rd, _RPD)],
                        den_out.at[c, pl.ds(rd, _RPD), :])

        @pl.when(s == 15)
        def _():
            pltpu.sync_copy(num_acc.at[pl.ds(16 * _RPN, 88)],
                            num_out.at[c, pl.ds(16 * _RPN, 88), :])
            pltpu.sync_copy(den_acc.at[pl.ds(16 * _RPD, 16)],
                            den_out.at[c, pl.ds(16 * _RPD, 16), :])

    return k(src_tab, q_tab, src_idx, dst_idx, zrows)


def _edge_op(src_tab, q_tab, src, dst, zrows):
    num_p, den_p = _edge_sc_kernel(src_tab, q_tab, src, dst, zrows)
    num0 = num_p[0][:_N // 2].reshape(_N, 64)
    num1 = num_p[1][:_N // 2].reshape(_N, 64)
    den0 = den_p[0][:782].reshape(782 * 32, 4)[:_N]
    den1 = den_p[1][:782].reshape(782 * 32, 4)[:_N]
    num = jnp.concatenate([num0, num1], axis=1)
    den = jnp.concatenate([den0, den1], axis=1)
    return num, den


def _bd(w):
    out = jnp.zeros((_H, _H), w.dtype)
    for h in range(_HEADS):
        out = out.at[h * _HD:(h + 1) * _HD, h * _HD:(h + 1) * _HD].set(w[h])
    return out


def kernel(x_breaker, x_bus, params, edge_index_bus_breaker,
           edge_index_breaker_bus):
    p = params
    src_bb = edge_index_bus_breaker[0].astype(jnp.int32)
    dst_bb = edge_index_bus_breaker[1].astype(jnp.int32)
    src_brb = edge_index_breaker_bus[0].astype(jnp.int32)
    dst_brb = edge_index_breaker_bus[1].astype(jnp.int32)

    ek = {'bus': 'bus__connects__breaker', 'breaker': 'breaker__connects__bus'}

    def fused_kv(l, t):
        key = f'{l}_{ek[t]}'
        bd_att = _bd(p['W_att_' + key])
        bd_msg = _bd(p['W_msg_' + key])
        pscale = jnp.repeat(p['p_rel_' + key] * _INV_SQRT_HD, _HD)
        kw = (p[f'K_w_{l}_{t}'] @ bd_att) * pscale[None, :]
        kb = (p[f'K_b_{l}_{t}'] @ bd_att) * pscale
        vw = p[f'V_w_{l}_{t}'] @ bd_msg
        vb = p[f'V_b_{l}_{t}'] @ bd_msg
        wcat = jnp.concatenate([kw[:, :64], vw[:, :64], kw[:, 64:], vw[:, 64:]], 1)
        bcat = jnp.concatenate([kb[:64], vb[:64], kb[64:], vb[64:]], 0)
        return wcat, bcat

    zrows = jnp.zeros((_RPN, 128), jnp.float32)

    h = {t: _proj(x_breaker if t == 'breaker' else x_bus,
                  p['lin_w_' + t], p['lin_b_' + t], out_relu=True)
         for t in _TYPES}

    tabs = {}
    for t in _TYPES:
        kvw, kvb = fused_kv(0, t)
        wcat = jnp.concatenate([kvw, p[f'Q_w_0_{t}']], axis=1)
        bcat = jnp.concatenate([kvb, p[f'Q_b_0_{t}']], axis=0)
        src_tab, q_tab = _proj(h[t], wcat, bcat, want_src=True, want_q=True)
        tabs[t] = (src_tab.reshape(2 * _N, 128), q_tab)

    num_br, den_br = _edge_op(tabs['bus'][0], tabs['breaker'][1],
                              src_bb, dst_bb, zrows)
    num_bus, den_bus = _edge_op(tabs['breaker'][0], tabs['bus'][1],
                                src_brb, dst_brb, zrows)

    beta = {t: jax.nn.sigmoid(p[f'skip_0_{t}']) for t in _TYPES}
    t_br, s_br, ss_br = _post(num_br, den_br, h['breaker'],
                              p['A_w_0_breaker'], p['A_b_0_breaker'],
                              beta['breaker'])
    t_bus, s_bus, ss_bus = _post(num_bus, den_bus, h['bus'],
                                 p['A_w_0_bus'], p['A_b_0_bus'], beta['bus'])
    sc_br, sh_br = _bn_affine(s_br, ss_br, p['bn_g_0_breaker'], p['bn_b_0_breaker'])
    sc_bus, sh_bus = _bn_affine(s_bus, ss_bus, p['bn_g_0_bus'], p['bn_b_0_bus'])

    q1_tab, x1_br = _proj(t_br, p['Q_w_1_breaker'], p['Q_b_1_breaker'],
                          scale=sc_br, shift=sh_br, want_q=True, emit_x=True)
    kvw, kvb = fused_kv(1, 'bus')
    src1_tab = _proj(t_bus, kvw, kvb, scale=sc_bus, shift=sh_bus,
                     want_src=True)

    num1, den1 = _edge_op(src1_tab.reshape(2 * _N, 128), q1_tab,
                          src_bb, dst_bb, zrows)

    beta1 = jax.nn.sigmoid(p['skip_1_breaker'])
    t1_br, s1, ss1 = _post(num1, den1, x1_br, p['A_w_1_breaker'],
                           p['A_b_1_breaker'], beta1)
    sc1, sh1 = _bn_affine(s1, ss1, p['bn_g_1_breaker'], p['bn_b_1_breaker'])

    out = _proj(t1_br, p['final_w'], p['final_b'], scale=sc1, shift=sh1)
    return out
